# baseline (speedup 1.0000x reference)
import jax
import jax.numpy as jnp
from jax.experimental import pallas as pl
from jax.experimental.pallas import tpu as pltpu

_TM = 400


def _elu(x):
    return jnp.where(x > 0.0, x, jnp.exp(x) - 1.0)


def _mega_body(pos_ref, w0_ref, adj_ref, wtop_ref, wpass_ref, bl_ref,
               bp_ref, ml_ref, mp_ref, sfull, lbf):
    nl = pl.num_programs(0)
    l = pl.program_id(0)
    i = pl.program_id(1)

    @pl.when((l == 0) & (i == 0))
    def _():
        sfull[...] = jnp.dot(pos_ref[...], w0_ref[...],
                             preferred_element_type=jnp.float32)

    @pl.when(i == 0)
    def _():
        lbf[...] = sfull[:, :lbf.shape[1]].astype(jnp.bfloat16)

    agg = jnp.dot(adj_ref[...], lbf[...], preferred_element_type=jnp.float32)
    xl = _elu(agg + bl_ref[0])
    sfi = sfull[pl.ds(i * _TM, _TM), :]
    yp = _elu(sfi + bp_ref[0])

    @pl.when(l < nl - 1)
    def _():
        s_next = (jnp.dot(xl, wtop_ref[0], preferred_element_type=jnp.float32)
                  + jnp.dot(yp, wpass_ref[0], preferred_element_type=jnp.float32))
        sfull[pl.ds(i * _TM, _TM), :] = s_next

    pml = jnp.max(xl, axis=0, keepdims=True)
    pmp = jnp.max(yp, axis=0, keepdims=True)

    @pl.when((l == nl - 1) & (i == 0))
    def _():
        ml_ref[...] = pml
        mp_ref[...] = pmp

    @pl.when((l == nl - 1) & (i > 0))
    def _():
        ml_ref[...] = jnp.maximum(ml_ref[...], pml)
        mp_ref[...] = jnp.maximum(mp_ref[...], pmp)


def _pad_up(v, m):
    return (v + m - 1) // m * m


def kernel(positions, adj, Ws, bs):
    n, fin0 = positions.shape
    adj_bf = adj.astype(jnp.bfloat16)
    fouts = [W.shape[1] for W in Ws]
    sls = [max(f // 3, 2) for f in fouts]
    nl = len(Ws)
    num_tiles = n // _TM

    lpad = _pad_up(max(sls), 128)
    fpad = _pad_up(max(fouts), 128)

    w0p = jnp.zeros((fin0, fpad), jnp.float32).at[:, :fouts[0]].set(Ws[0])
    wtop = jnp.zeros((nl, lpad, fpad), jnp.float32)
    wpass = jnp.zeros((nl, fpad, fpad), jnp.float32)
    blv = jnp.zeros((nl, lpad), jnp.float32)
    bpv = jnp.zeros((nl, fpad), jnp.float32)
    for L in range(nl):
        sl, fo = sls[L], fouts[L]
        blv = blv.at[L, :sl].set(bs[L][:sl])
        bpv = bpv.at[L, sl:fo].set(bs[L][sl:])
        if L + 1 < nl:
            fn = fouts[L + 1]
            wtop = wtop.at[L, :sl, :fn].set(Ws[L + 1][:sl])
            wpass = wpass.at[L, sl:fo, :fn].set(Ws[L + 1][sl:])

    ml, mp = pl.pallas_call(
        _mega_body,
        grid=(nl, num_tiles),
        in_specs=[
            pl.BlockSpec((n, fin0), lambda l, i: (0, 0)),
            pl.BlockSpec((fin0, fpad), lambda l, i: (0, 0)),
            pl.BlockSpec((_TM, n), lambda l, i: (i, 0)),
            pl.BlockSpec((1, lpad, fpad), lambda l, i: (l, 0, 0)),
            pl.BlockSpec((1, fpad, fpad), lambda l, i: (l, 0, 0)),
            pl.BlockSpec((1, 1, lpad), lambda l, i: (l, 0, 0)),
            pl.BlockSpec((1, 1, fpad), lambda l, i: (l, 0, 0)),
        ],
        out_specs=[
            pl.BlockSpec((1, lpad), lambda l, i: (0, 0)),
            pl.BlockSpec((1, fpad), lambda l, i: (0, 0)),
        ],
        out_shape=[
            jax.ShapeDtypeStruct((1, lpad), jnp.float32),
            jax.ShapeDtypeStruct((1, fpad), jnp.float32),
        ],
        scratch_shapes=[
            pltpu.VMEM((n, fpad), jnp.float32),
            pltpu.VMEM((n, lpad), jnp.bfloat16),
        ],
    )(positions, w0p, adj_bf, wtop, wpass,
      blv.reshape(nl, 1, lpad), bpv.reshape(nl, 1, fpad))

    sl_last, fo_last = sls[-1], fouts[-1]
    return jnp.concatenate([ml[0, :sl_last], mp[0, sl_last:fo_last]], axis=0)


# speedup vs baseline: 1.0924x; 1.0924x over previous
import jax
import jax.numpy as jnp
from jax.experimental import pallas as pl

_TM = 1000
_TM0 = 400


def _elu(x):
    return jnp.where(x > 0.0, x, jnp.exp(x) - 1.0)


def _first_body(pos_ref, w_ref, ol_ref, or_ref):
    s = jnp.dot(pos_ref[...], w_ref[...], preferred_element_type=jnp.float32)
    sl = ol_ref.shape[1]
    ol_ref[...] = s[:, :sl].astype(jnp.bfloat16)
    or_ref[...] = s[:, sl:]


def _tail(agg, sright_ref, bl_ref, br_ref, wtop_ref, wbot_ref, ol_ref, or_ref):
    xl = _elu(agg + bl_ref[...])
    xr = _elu(sright_ref[...] + br_ref[...])
    s = (jnp.dot(xl, wtop_ref[...], preferred_element_type=jnp.float32)
         + jnp.dot(xr, wbot_ref[...], preferred_element_type=jnp.float32))
    sln = ol_ref.shape[1]
    ol_ref[...] = s[:, :sln].astype(jnp.bfloat16)
    or_ref[...] = s[:, sln:]


def _l0_body(adj_ref, sleft_ref, sright_ref, bl_ref, br_ref, wtop_ref,
             wbot_ref, adjb_ref, ol_ref, or_ref):
    ab = adj_ref[...].astype(jnp.bfloat16)
    adjb_ref[...] = ab
    agg = jnp.dot(ab, sleft_ref[...], preferred_element_type=jnp.float32)
    _tail(agg, sright_ref, bl_ref, br_ref, wtop_ref, wbot_ref, ol_ref, or_ref)


def _mid_body(adjb_ref, sleft_ref, sright_ref, bl_ref, br_ref, wtop_ref,
              wbot_ref, ol_ref, or_ref):
    agg = jnp.dot(adjb_ref[...], sleft_ref[...],
                  preferred_element_type=jnp.float32)
    _tail(agg, sright_ref, bl_ref, br_ref, wtop_ref, wbot_ref, ol_ref, or_ref)


def _last_body(adjb_ref, sleft_ref, sright_ref, bl_ref, br_ref,
               ml_ref, mr_ref):
    i = pl.program_id(0)
    agg = jnp.dot(adjb_ref[...], sleft_ref[...],
                  preferred_element_type=jnp.float32)
    xl = _elu(agg + bl_ref[...])
    xr = _elu(sright_ref[...] + br_ref[...])
    pml = jnp.max(xl, axis=0, keepdims=True)
    pmr = jnp.max(xr, axis=0, keepdims=True)

    @pl.when(i == 0)
    def _():
        ml_ref[...] = pml
        mr_ref[...] = pmr

    @pl.when(i > 0)
    def _():
        ml_ref[...] = jnp.maximum(ml_ref[...], pml)
        mr_ref[...] = jnp.maximum(mr_ref[...], pmr)


def kernel(positions, adj, Ws, bs):
    n, fin0 = positions.shape
    fouts = [W.shape[1] for W in Ws]
    sls = [max(f // 3, 2) for f in fouts]
    nl = len(Ws)
    f0, s0 = fouts[0], sls[0]

    sleft, sright = pl.pallas_call(
        _first_body,
        grid=(1,),
        in_specs=[
            pl.BlockSpec((n, fin0), lambda i: (0, 0)),
            pl.BlockSpec((fin0, f0), lambda i: (0, 0)),
        ],
        out_specs=[
            pl.BlockSpec((n, s0), lambda i: (0, 0)),
            pl.BlockSpec((n, f0 - s0), lambda i: (0, 0)),
        ],
        out_shape=[
            jax.ShapeDtypeStruct((n, s0), jnp.bfloat16),
            jax.ShapeDtypeStruct((n, f0 - s0), jnp.float32),
        ],
    )(positions, Ws[0])

    adjb = None
    for L in range(nl - 1):
        sl, fout = sls[L], fouts[L]
        wr = fout - sl
        sln, fn = sls[L + 1], fouts[L + 1]
        b = bs[L].reshape(1, fout)
        bl, br = b[:, :sl], b[:, sl:]
        wtop, wbot = Ws[L + 1][:sl], Ws[L + 1][sl:]
        tm = _TM0 if L == 0 else _TM
        in_specs = [
            pl.BlockSpec((tm, n), lambda i: (i, 0)),
            pl.BlockSpec((n, sl), lambda i: (0, 0)),
            pl.BlockSpec((tm, wr), lambda i: (i, 0)),
            pl.BlockSpec((1, sl), lambda i: (0, 0)),
            pl.BlockSpec((1, wr), lambda i: (0, 0)),
            pl.BlockSpec((sl, fn), lambda i: (0, 0)),
            pl.BlockSpec((wr, fn), lambda i: (0, 0)),
        ]
        out_specs = [
            pl.BlockSpec((tm, sln), lambda i: (i, 0)),
            pl.BlockSpec((tm, fn - sln), lambda i: (i, 0)),
        ]
        out_shape = [
            jax.ShapeDtypeStruct((n, sln), jnp.bfloat16),
            jax.ShapeDtypeStruct((n, fn - sln), jnp.float32),
        ]
        if L == 0:
            adjb, sleft, sright = pl.pallas_call(
                _l0_body,
                grid=(n // _TM0,),
                in_specs=in_specs,
                out_specs=[pl.BlockSpec((_TM0, n), lambda i: (i, 0))] + out_specs,
                out_shape=[jax.ShapeDtypeStruct((n, n), jnp.bfloat16)] + out_shape,
            )(adj, sleft, sright, bl, br, wtop, wbot)
        else:
            sleft, sright = pl.pallas_call(
                _mid_body,
                grid=(n // _TM,),
                in_specs=in_specs,
                out_specs=out_specs,
                out_shape=out_shape,
            )(adjb, sleft, sright, bl, br, wtop, wbot)

    sl, fout = sls[-1], fouts[-1]
    wr = fout - sl
    b = bs[-1].reshape(1, fout)
    bl, br = b[:, :sl], b[:, sl:]
    ml, mr = pl.pallas_call(
        _last_body,
        grid=(n // _TM,),
        in_specs=[
            pl.BlockSpec((_TM, n), lambda i: (i, 0)),
            pl.BlockSpec((n, sl), lambda i: (0, 0)),
            pl.BlockSpec((_TM, wr), lambda i: (i, 0)),
            pl.BlockSpec((1, sl), lambda i: (0, 0)),
            pl.BlockSpec((1, wr), lambda i: (0, 0)),
        ],
        out_specs=[
            pl.BlockSpec((1, sl), lambda i: (0, 0)),
            pl.BlockSpec((1, wr), lambda i: (0, 0)),
        ],
        out_shape=[
            jax.ShapeDtypeStruct((1, sl), jnp.float32),
            jax.ShapeDtypeStruct((1, wr), jnp.float32),
        ],
    )(adjb, sleft, sright, bl, br)

    return jnp.concatenate([ml[0], mr[0]], axis=0)


# speedup vs baseline: 1.1578x; 1.0599x over previous
import jax
import jax.numpy as jnp
from jax.experimental import pallas as pl

_TM = 1000
_TM0 = 400
_H1 = 5120


def _elu(x):
    return jnp.where(x > 0.0, x, jnp.exp(x) - 1.0)


def _first_body(pos_ref, w_ref, ol_ref, or_ref):
    s = jnp.dot(pos_ref[...], w_ref[...], preferred_element_type=jnp.float32)
    sl = ol_ref.shape[1]
    ol_ref[...] = s[:, :sl].astype(jnp.bfloat16)
    or_ref[...] = s[:, sl:]


def _tail(agg, sright_ref, bl_ref, br_ref, wtop_ref, wbot_ref, ol_ref, or_ref):
    xl = _elu(agg + bl_ref[...])
    xr = _elu(sright_ref[...] + br_ref[...])
    s = (jnp.dot(xl, wtop_ref[...], preferred_element_type=jnp.float32)
         + jnp.dot(xr, wbot_ref[...], preferred_element_type=jnp.float32))
    sln = ol_ref.shape[1]
    ol_ref[...] = s[:, :sln].astype(jnp.bfloat16)
    or_ref[...] = s[:, sln:]


def _agg2(ab1_ref, ab2_ref, sleft_ref):
    h1 = ab1_ref.shape[1]
    return (jnp.dot(ab1_ref[...], sleft_ref[:h1, :],
                    preferred_element_type=jnp.float32)
            + jnp.dot(ab2_ref[...], sleft_ref[h1:, :],
                      preferred_element_type=jnp.float32))


def _l0_body(adj_ref, sleft_ref, sright_ref, bl_ref, br_ref, wtop_ref,
             wbot_ref, ab1_ref, ab2_ref, ol_ref, or_ref):
    h1 = ab1_ref.shape[1]
    ab = adj_ref[...].astype(jnp.bfloat16)
    ab1_ref[...] = ab[:, :h1]
    ab2_ref[...] = ab[:, h1:]
    agg = jnp.dot(ab, sleft_ref[...], preferred_element_type=jnp.float32)
    _tail(agg, sright_ref, bl_ref, br_ref, wtop_ref, wbot_ref, ol_ref, or_ref)


def _mid_body(ab1_ref, ab2_ref, sleft_ref, sright_ref, bl_ref, br_ref,
              wtop_ref, wbot_ref, ol_ref, or_ref):
    agg = _agg2(ab1_ref, ab2_ref, sleft_ref)
    _tail(agg, sright_ref, bl_ref, br_ref, wtop_ref, wbot_ref, ol_ref, or_ref)


def _last_body(ab1_ref, ab2_ref, sleft_ref, sright_ref, bl_ref, br_ref,
               ml_ref, mr_ref):
    i = pl.program_id(0)
    agg = _agg2(ab1_ref, ab2_ref, sleft_ref)
    xl = _elu(agg + bl_ref[...])
    xr = _elu(sright_ref[...] + br_ref[...])
    pml = jnp.max(xl, axis=0, keepdims=True)
    pmr = jnp.max(xr, axis=0, keepdims=True)

    @pl.when(i == 0)
    def _():
        ml_ref[...] = pml
        mr_ref[...] = pmr

    @pl.when(i > 0)
    def _():
        ml_ref[...] = jnp.maximum(ml_ref[...], pml)
        mr_ref[...] = jnp.maximum(mr_ref[...], pmr)


def kernel(positions, adj, Ws, bs):
    n, fin0 = positions.shape
    fouts = [W.shape[1] for W in Ws]
    sls = [max(f // 3, 2) for f in fouts]
    nl = len(Ws)
    f0, s0 = fouts[0], sls[0]
    h1 = min(_H1, n)
    h2 = n - h1

    sleft, sright = pl.pallas_call(
        _first_body,
        grid=(1,),
        in_specs=[
            pl.BlockSpec((n, fin0), lambda i: (0, 0)),
            pl.BlockSpec((fin0, f0), lambda i: (0, 0)),
        ],
        out_specs=[
            pl.BlockSpec((n, s0), lambda i: (0, 0)),
            pl.BlockSpec((n, f0 - s0), lambda i: (0, 0)),
        ],
        out_shape=[
            jax.ShapeDtypeStruct((n, s0), jnp.bfloat16),
            jax.ShapeDtypeStruct((n, f0 - s0), jnp.float32),
        ],
    )(positions, Ws[0])

    ab1 = ab2 = None
    for L in range(nl - 1):
        sl, fout = sls[L], fouts[L]
        wr = fout - sl
        sln, fn = sls[L + 1], fouts[L + 1]
        b = bs[L].reshape(1, fout)
        bl, br = b[:, :sl], b[:, sl:]
        wtop, wbot = Ws[L + 1][:sl], Ws[L + 1][sl:]
        tm = _TM0 if L == 0 else _TM
        common_specs = [
            pl.BlockSpec((n, sl), lambda i: (0, 0)),
            pl.BlockSpec((tm, wr), lambda i: (i, 0)),
            pl.BlockSpec((1, sl), lambda i: (0, 0)),
            pl.BlockSpec((1, wr), lambda i: (0, 0)),
            pl.BlockSpec((sl, fn), lambda i: (0, 0)),
            pl.BlockSpec((wr, fn), lambda i: (0, 0)),
        ]
        out_specs = [
            pl.BlockSpec((tm, sln), lambda i: (i, 0)),
            pl.BlockSpec((tm, fn - sln), lambda i: (i, 0)),
        ]
        out_shape = [
            jax.ShapeDtypeStruct((n, sln), jnp.bfloat16),
            jax.ShapeDtypeStruct((n, fn - sln), jnp.float32),
        ]
        if L == 0:
            ab1, ab2, sleft, sright = pl.pallas_call(
                _l0_body,
                grid=(n // _TM0,),
                in_specs=[pl.BlockSpec((tm, n), lambda i: (i, 0))] + common_specs,
                out_specs=[pl.BlockSpec((_TM0, h1), lambda i: (i, 0)),
                           pl.BlockSpec((_TM0, h2), lambda i: (i, 0))] + out_specs,
                out_shape=[jax.ShapeDtypeStruct((n, h1), jnp.bfloat16),
                           jax.ShapeDtypeStruct((n, h2), jnp.bfloat16)] + out_shape,
            )(adj, sleft, sright, bl, br, wtop, wbot)
        else:
            sleft, sright = pl.pallas_call(
                _mid_body,
                grid=(n // _TM,),
                in_specs=[pl.BlockSpec((tm, h1), lambda i: (i, 0)),
                          pl.BlockSpec((tm, h2), lambda i: (i, 0))] + common_specs,
                out_specs=out_specs,
                out_shape=out_shape,
            )(ab1, ab2, sleft, sright, bl, br, wtop, wbot)

    sl, fout = sls[-1], fouts[-1]
    wr = fout - sl
    b = bs[-1].reshape(1, fout)
    bl, br = b[:, :sl], b[:, sl:]
    ml, mr = pl.pallas_call(
        _last_body,
        grid=(n // _TM,),
        in_specs=[
            pl.BlockSpec((_TM, h1), lambda i: (i, 0)),
            pl.BlockSpec((_TM, h2), lambda i: (i, 0)),
            pl.BlockSpec((n, sl), lambda i: (0, 0)),
            pl.BlockSpec((_TM, wr), lambda i: (i, 0)),
            pl.BlockSpec((1, sl), lambda i: (0, 0)),
            pl.BlockSpec((1, wr), lambda i: (0, 0)),
        ],
        out_specs=[
            pl.BlockSpec((1, sl), lambda i: (0, 0)),
            pl.BlockSpec((1, wr), lambda i: (0, 0)),
        ],
        out_shape=[
            jax.ShapeDtypeStruct((1, sl), jnp.float32),
            jax.ShapeDtypeStruct((1, wr), jnp.float32),
        ],
    )(ab1, ab2, sleft, sright, bl, br)

    return jnp.concatenate([ml[0], mr[0]], axis=0)
